# baseline (speedup 1.0000x reference)
import jax
import jax.numpy as jnp
from jax.experimental import pallas as pl
from jax.experimental.pallas import tpu as pltpu

B = 8
N = 8192
F = 512
L = 500
D = 128
NUM_CLASSES = 2
BUFFER = 1024

T = 1024
NT = N // T
STEPS = B * NT
RB = BUFFER // STEPS

NEG = -1e30


def _fused_kernel(x_ref, W1_ref, b1_ref, Wa1_ref, ba1_ref, Wa2_ref, ba2_ref,
                  Wc_ref, bc_ref, rehA_ref, rehB_ref,
                  logits_ref, newreh_ref,
                  acc_ref, m_ref, s_ref, bf_ref, Hs_ref, Hbs_ref):
    b = pl.program_id(0)
    n = pl.program_id(1)
    t = b * NT + n
    slot = jax.lax.rem(t, 2)
    prev = jax.lax.rem(t + 1, 2)

    @pl.when(t == 0)
    def _init_carry():
        Hs_ref[1] = jnp.zeros((T, L), jnp.float32)
        Hbs_ref[1] = jnp.zeros((T, L), jnp.bfloat16)

    @pl.when(n == 0)
    def _init_bag():
        m_ref[0, 0] = NEG
        s_ref[0, 0] = 0.0
        acc_ref[...] = jnp.zeros_like(acc_ref)

    def _attend(Hp, Hbp, p_scale, mask_scores):
        a = jnp.maximum(jnp.dot(Hbp, Wa1_ref[...],
                                preferred_element_type=jnp.float32)
                        + ba1_ref[0], 0.0)
        sc = jnp.dot(a, Wa2_ref[...],
                     preferred_element_type=jnp.float32) + ba2_ref[0]
        if mask_scores:
            sc = jnp.where(p_scale > 0.0, sc, NEG)
        m_old = m_ref[0, 0]
        m_new = jnp.maximum(m_old, jnp.max(sc))
        corr = jnp.exp(m_old - m_new)
        p = jnp.exp(sc - m_new)
        if mask_scores:
            p = p * p_scale
        m_ref[0, 0] = m_new
        s_ref[0, 0] = s_ref[0, 0] * corr + jnp.sum(p)
        acc_ref[...] = acc_ref[...] * corr + jax.lax.dot_general(
            p, Hp, (((0,), (0,)), ((), ())),
            preferred_element_type=jnp.float32).reshape(1, L)

    Hp = Hs_ref[pl.ds(prev, 1), :, :][0]
    Hbp = Hbs_ref[pl.ds(prev, 1), :, :][0]
    valid = (n > 0).astype(jnp.float32)
    _attend(Hp, Hbp, valid, True)

    x_t = x_ref[0]
    H = jnp.maximum(jnp.dot(x_t.astype(jnp.bfloat16), W1_ref[...],
                            preferred_element_type=jnp.float32)
                    + b1_ref[0], 0.0)
    Hb = H.astype(jnp.bfloat16)
    Hs_ref[pl.ds(slot, 1), :, :] = H[None]
    Hbs_ref[pl.ds(slot, 1), :, :] = Hb[None]

    @pl.when(n == NT - 1)
    def _finish_bag():
        _attend(H, Hb, 1.0, False)
        bf = acc_ref[...] / s_ref[0, 0]
        bf_ref[pl.ds(b, 1), :] = bf
        logits_ref[0] = jnp.dot(bf, Wc_ref[...],
                                preferred_element_type=jnp.float32) + bc_ref[...]

    @pl.when(t < STEPS - 1)
    def _copy_reh():
        newreh_ref[0:B, :] = rehA_ref[...]

    @pl.when(t == STEPS - 1)
    def _write_bagfeats():
        newreh_ref[0:B, :] = bf_ref[...]

    newreh_ref[B:2 * B, :] = rehB_ref[...]


def kernel(x, W1, b1, Wa1, ba1, Wa2, ba2, Wc, bc, rehearsal):
    reh_flat = rehearsal.reshape(BUFFER, L)

    grid = (B, NT)
    in_specs = [
        pl.BlockSpec((1, T, F), lambda b, n: (b, n, 0)),
        pl.BlockSpec((F, L), lambda b, n: (0, 0)),
        pl.BlockSpec((1, L), lambda b, n: (0, 0)),
        pl.BlockSpec((L, D), lambda b, n: (0, 0)),
        pl.BlockSpec((1, D), lambda b, n: (0, 0)),
        pl.BlockSpec((D, 1), lambda b, n: (0, 0)),
        pl.BlockSpec((1, 1), lambda b, n: (0, 0)),
        pl.BlockSpec((L, NUM_CLASSES), lambda b, n: (0, 0)),
        pl.BlockSpec((1, NUM_CLASSES), lambda b, n: (0, 0)),
        pl.BlockSpec((B, L), lambda b, n: (2 * (b * NT + n) + 1, 0)),
        pl.BlockSpec((B, L), lambda b, n: ((2 * (b * NT + n) + 2) % (BUFFER // B), 0)),
    ]
    out_specs = [
        pl.BlockSpec((1, 1, NUM_CLASSES), lambda b, n: (b, 0, 0)),
        pl.BlockSpec((RB, L), lambda b, n: ((b * NT + n + 1) % STEPS, 0)),
    ]
    out_shapes = [
        jax.ShapeDtypeStruct((B, 1, NUM_CLASSES), jnp.float32),
        jax.ShapeDtypeStruct((BUFFER, L), jnp.float32),
    ]
    scratch_shapes = [
        pltpu.VMEM((1, L), jnp.float32),
        pltpu.SMEM((1, 1), jnp.float32),
        pltpu.SMEM((1, 1), jnp.float32),
        pltpu.VMEM((B, L), jnp.float32),
        pltpu.VMEM((2, T, L), jnp.float32),
        pltpu.VMEM((2, T, L), jnp.bfloat16),
    ]

    logits, newreh = pl.pallas_call(
        _fused_kernel,
        grid=grid,
        in_specs=in_specs,
        out_specs=out_specs,
        out_shape=out_shapes,
        scratch_shapes=scratch_shapes,
        compiler_params=pltpu.CompilerParams(
            dimension_semantics=("arbitrary", "arbitrary"),
        ),
    )(x, W1.astype(jnp.bfloat16), b1.reshape(1, L),
      Wa1.astype(jnp.bfloat16), ba1.reshape(1, D), Wa2,
      ba2.reshape(1, 1), Wc, bc.reshape(1, NUM_CLASSES), reh_flat, reh_flat)

    return (logits.reshape(B, NUM_CLASSES),
            newreh.reshape(NUM_CLASSES, BUFFER // NUM_CLASSES, L))


# speedup vs baseline: 1.0646x; 1.0646x over previous
import jax
import jax.numpy as jnp
from jax.experimental import pallas as pl
from jax.experimental.pallas import tpu as pltpu

B = 8
N = 8192
F = 512
L = 500
D = 128
NUM_CLASSES = 2
BUFFER = 1024

T = 1024
NT = N // T
STEPS = B * NT
RB = BUFFER // STEPS

NEG = -1e30


def _fused_kernel(x_ref, W1_ref, b1_ref, Wa1_ref, ba1_ref, Wa2_ref, ba2_ref,
                  Wc_ref, bc_ref, rehA_ref, rehB_ref,
                  logits_ref, newreh_ref,
                  acc_ref, m_ref, s_ref, bf_ref, Hs_ref, Hbs_ref):
    t = pl.program_id(0)
    tmod = jax.lax.rem(t, NT)
    slot = jax.lax.rem(t, 2)
    prev = jax.lax.rem(t + 1, 2)

    @pl.when(tmod == 1)
    def _init_bag():
        m_ref[0, 0] = NEG
        s_ref[0, 0] = 0.0
        acc_ref[...] = jnp.zeros_like(acc_ref)

    @pl.when(t > 0)
    def _attend():
        Hp = Hs_ref[pl.ds(prev, 1), :, :][0]
        Hbp = Hbs_ref[pl.ds(prev, 1), :, :][0]
        a = jnp.maximum(jnp.dot(Hbp, Wa1_ref[...],
                                preferred_element_type=jnp.float32)
                        + ba1_ref[0], 0.0)
        sc = jnp.dot(a, Wa2_ref[...],
                     preferred_element_type=jnp.float32) + ba2_ref[0]
        m_old = m_ref[0, 0]
        m_new = jnp.maximum(m_old, jnp.max(sc))
        corr = jnp.exp(m_old - m_new)
        p = jnp.exp(sc - m_new)
        m_ref[0, 0] = m_new
        s_ref[0, 0] = s_ref[0, 0] * corr + jnp.sum(p)
        acc_ref[...] = acc_ref[...] * corr + jax.lax.dot_general(
            p, Hp, (((0,), (0,)), ((), ())),
            preferred_element_type=jnp.float32).reshape(1, L)

    @pl.when((tmod == 0) & (t > 0))
    def _finish_bag():
        bb = t // NT - 1
        bf = acc_ref[...] / s_ref[0, 0]
        bf_ref[pl.ds(bb, 1), :] = bf
        logits_ref[0] = jnp.dot(bf, Wc_ref[...],
                                preferred_element_type=jnp.float32) + bc_ref[...]

    @pl.when(t < STEPS)
    def _produce():
        x_t = x_ref[0]
        H = jnp.maximum(jnp.dot(x_t.astype(jnp.bfloat16), W1_ref[...],
                                preferred_element_type=jnp.float32)
                        + b1_ref[0], 0.0)
        Hs_ref[pl.ds(slot, 1), :, :] = H[None]
        Hbs_ref[pl.ds(slot, 1), :, :] = H.astype(jnp.bfloat16)[None]

    @pl.when(t < STEPS)
    def _copy_reh():
        newreh_ref[0:B, :] = rehA_ref[...]

    @pl.when(t == STEPS)
    def _write_bagfeats():
        newreh_ref[0:B, :] = bf_ref[...]

    newreh_ref[B:2 * B, :] = rehB_ref[...]


def kernel(x, W1, b1, Wa1, ba1, Wa2, ba2, Wc, bc, rehearsal):
    reh_flat = rehearsal.reshape(BUFFER, L)

    grid = (STEPS + 1,)

    def _x_map(t):
        tc = jnp.minimum(t, STEPS - 1)
        return (tc // NT, tc % NT, 0)

    in_specs = [
        pl.BlockSpec((1, T, F), _x_map),
        pl.BlockSpec((F, L), lambda t: (0, 0)),
        pl.BlockSpec((1, L), lambda t: (0, 0)),
        pl.BlockSpec((L, D), lambda t: (0, 0)),
        pl.BlockSpec((1, D), lambda t: (0, 0)),
        pl.BlockSpec((D, 1), lambda t: (0, 0)),
        pl.BlockSpec((1, 1), lambda t: (0, 0)),
        pl.BlockSpec((L, NUM_CLASSES), lambda t: (0, 0)),
        pl.BlockSpec((1, NUM_CLASSES), lambda t: (0, 0)),
        pl.BlockSpec((B, L),
                     lambda t: (jnp.where(t < STEPS - 1, 2 * t + 1, 125), 0)),
        pl.BlockSpec((B, L),
                     lambda t: (jnp.where(t < STEPS - 1, 2 * t + 2,
                                          jnp.where(t == STEPS - 1, 126, 0)), 0)),
    ]
    out_specs = [
        pl.BlockSpec((1, 1, NUM_CLASSES),
                     lambda t: (jnp.clip(t // NT - 1, 0, B - 1), 0, 0)),
        pl.BlockSpec((RB, L),
                     lambda t: (jnp.where(t < STEPS - 1, t + 1,
                                          jnp.where(t == STEPS - 1, STEPS - 1, 0)),
                                0)),
    ]
    out_shapes = [
        jax.ShapeDtypeStruct((B, 1, NUM_CLASSES), jnp.float32),
        jax.ShapeDtypeStruct((BUFFER, L), jnp.float32),
    ]
    scratch_shapes = [
        pltpu.VMEM((1, L), jnp.float32),
        pltpu.SMEM((1, 1), jnp.float32),
        pltpu.SMEM((1, 1), jnp.float32),
        pltpu.VMEM((B, L), jnp.float32),
        pltpu.VMEM((2, T, L), jnp.float32),
        pltpu.VMEM((2, T, L), jnp.bfloat16),
    ]

    logits, newreh = pl.pallas_call(
        _fused_kernel,
        grid=grid,
        in_specs=in_specs,
        out_specs=out_specs,
        out_shape=out_shapes,
        scratch_shapes=scratch_shapes,
        compiler_params=pltpu.CompilerParams(
            dimension_semantics=("arbitrary",),
        ),
    )(x, W1.astype(jnp.bfloat16), b1.reshape(1, L),
      Wa1.astype(jnp.bfloat16), ba1.reshape(1, D), Wa2,
      ba2.reshape(1, 1), Wc, bc.reshape(1, NUM_CLASSES), reh_flat, reh_flat)

    return (logits.reshape(B, NUM_CLASSES),
            newreh.reshape(NUM_CLASSES, BUFFER // NUM_CLASSES, L))


# speedup vs baseline: 1.2532x; 1.1772x over previous
import jax
import jax.numpy as jnp
from jax.experimental import pallas as pl
from jax.experimental.pallas import tpu as pltpu

B = 8
N = 8192
F = 512
L = 500
D = 128
NUM_CLASSES = 2
BUFFER = 1024

T = 1024
NT = N // T
STEPS = B * NT
RB = BUFFER // STEPS


def _fused_kernel(x_ref, W1_ref, b1_ref, Wa1_ref, ba1_ref, Wa2t_ref,
                  Wc_ref, bc_ref, rehA_ref, rehB_ref,
                  logits_ref, newreh_ref,
                  acc_ref, m_ref, s_ref, bf_ref):
    b = pl.program_id(0)
    n = pl.program_id(1)
    t = b * NT + n

    @pl.when(n == 0)
    def _init():
        m_ref[0, 0] = -1e30
        s_ref[0, 0] = 0.0
        acc_ref[...] = jnp.zeros_like(acc_ref)

    x_t = x_ref[0]
    H = jnp.maximum(jnp.dot(x_t.astype(jnp.bfloat16), W1_ref[...],
                            preferred_element_type=jnp.float32)
                    + b1_ref[0], 0.0)
    Hb = H.astype(jnp.bfloat16)
    a = jnp.maximum(jnp.dot(Hb, Wa1_ref[...],
                            preferred_element_type=jnp.float32)
                    + ba1_ref[0], 0.0)
    sc = jax.lax.dot_general(Wa2t_ref[...], a, (((1,), (1,)), ((), ())),
                             preferred_element_type=jnp.float32)

    m_old = m_ref[0, 0]
    m_new = jnp.maximum(m_old, jnp.max(sc))
    corr = jnp.exp(m_old - m_new)
    p = jnp.exp(sc - m_new)
    m_ref[0, 0] = m_new
    s_ref[0, 0] = s_ref[0, 0] * corr + jnp.sum(p)
    acc_ref[...] = acc_ref[...] * corr + jnp.dot(
        p, H, preferred_element_type=jnp.float32)

    @pl.when(n == NT - 1)
    def _finish_bag():
        bf = acc_ref[...] / s_ref[0, 0]
        bf_ref[pl.ds(b, 1), :] = bf
        logits_ref[0] = jnp.dot(bf, Wc_ref[...],
                                preferred_element_type=jnp.float32) + bc_ref[...]

    @pl.when(t < STEPS - 1)
    def _copy_reh():
        newreh_ref[0:B, :] = rehA_ref[...]

    @pl.when(t == STEPS - 1)
    def _write_bagfeats():
        newreh_ref[0:B, :] = bf_ref[...]

    newreh_ref[B:2 * B, :] = rehB_ref[...]


def kernel(x, W1, b1, Wa1, ba1, Wa2, ba2, Wc, bc, rehearsal):
    reh_flat = rehearsal.reshape(BUFFER, L)

    grid = (B, NT)
    in_specs = [
        pl.BlockSpec((1, T, F), lambda b, n: (b, n, 0)),
        pl.BlockSpec((F, L), lambda b, n: (0, 0)),
        pl.BlockSpec((1, L), lambda b, n: (0, 0)),
        pl.BlockSpec((L, D), lambda b, n: (0, 0)),
        pl.BlockSpec((1, D), lambda b, n: (0, 0)),
        pl.BlockSpec((1, D), lambda b, n: (0, 0)),
        pl.BlockSpec((L, NUM_CLASSES), lambda b, n: (0, 0)),
        pl.BlockSpec((1, NUM_CLASSES), lambda b, n: (0, 0)),
        pl.BlockSpec((B, L), lambda b, n: (2 * (b * NT + n) + 1, 0)),
        pl.BlockSpec((B, L), lambda b, n: ((2 * (b * NT + n) + 2) % (BUFFER // B), 0)),
    ]
    out_specs = [
        pl.BlockSpec((1, 1, NUM_CLASSES), lambda b, n: (b, 0, 0)),
        pl.BlockSpec((RB, L), lambda b, n: ((b * NT + n + 1) % STEPS, 0)),
    ]
    out_shapes = [
        jax.ShapeDtypeStruct((B, 1, NUM_CLASSES), jnp.float32),
        jax.ShapeDtypeStruct((BUFFER, L), jnp.float32),
    ]
    scratch_shapes = [
        pltpu.VMEM((1, L), jnp.float32),
        pltpu.SMEM((1, 1), jnp.float32),
        pltpu.SMEM((1, 1), jnp.float32),
        pltpu.VMEM((B, L), jnp.float32),
    ]

    logits, newreh = pl.pallas_call(
        _fused_kernel,
        grid=grid,
        in_specs=in_specs,
        out_specs=out_specs,
        out_shape=out_shapes,
        scratch_shapes=scratch_shapes,
        compiler_params=pltpu.CompilerParams(
            dimension_semantics=("arbitrary", "arbitrary"),
        ),
    )(x, W1.astype(jnp.bfloat16), b1.reshape(1, L),
      Wa1.astype(jnp.bfloat16), ba1.reshape(1, D), Wa2.reshape(1, D),
      Wc, bc.reshape(1, NUM_CLASSES), reh_flat, reh_flat)

    return (logits.reshape(B, NUM_CLASSES),
            newreh.reshape(NUM_CLASSES, BUFFER // NUM_CLASSES, L))


# speedup vs baseline: 1.4169x; 1.1307x over previous
import jax
import jax.numpy as jnp
from jax.experimental import pallas as pl
from jax.experimental.pallas import tpu as pltpu

B = 8
N = 8192
F = 512
L = 500
D = 128
NUM_CLASSES = 2
BUFFER = 1024

T = 2048
NT = N // T
STEPS = B * NT
RB = BUFFER // STEPS
NV = RB // 8


def _fused_kernel(x_ref, W1_ref, b1_ref, Wa1_ref, ba1_ref, Wa2t_ref,
                  Wc_ref, bc_ref, *refs):
    reh_refs = refs[:NV]
    logits_ref, newreh_ref, acc_ref, m_ref, s_ref, bf_ref = refs[NV:]
    b = pl.program_id(0)
    n = pl.program_id(1)
    t = b * NT + n

    @pl.when(n == 0)
    def _init():
        m_ref[0, 0] = -1e30
        s_ref[0, 0] = 0.0
        acc_ref[...] = jnp.zeros_like(acc_ref)

    x_t = x_ref[0]
    H = jnp.maximum(jnp.dot(x_t.astype(jnp.bfloat16), W1_ref[...],
                            preferred_element_type=jnp.float32)
                    + b1_ref[0], 0.0)
    Hb = H.astype(jnp.bfloat16)
    a = jnp.maximum(jnp.dot(Hb, Wa1_ref[...],
                            preferred_element_type=jnp.float32)
                    + ba1_ref[0], 0.0)
    sc = jax.lax.dot_general(Wa2t_ref[...], a, (((1,), (1,)), ((), ())),
                             preferred_element_type=jnp.float32)

    m_old = m_ref[0, 0]
    m_new = jnp.maximum(m_old, jnp.max(sc))
    corr = jnp.exp(m_old - m_new)
    p = jnp.exp(sc - m_new)
    m_ref[0, 0] = m_new
    s_ref[0, 0] = s_ref[0, 0] * corr + jnp.sum(p)
    acc_ref[...] = acc_ref[...] * corr + jnp.dot(
        p, H, preferred_element_type=jnp.float32)

    @pl.when(n == NT - 1)
    def _finish_bag():
        bf = acc_ref[...] / s_ref[0, 0]
        bf_ref[pl.ds(b, 1), :] = bf
        logits_ref[0] = jnp.dot(bf, Wc_ref[...],
                                preferred_element_type=jnp.float32) + bc_ref[...]

    @pl.when(t < STEPS - 1)
    def _copy_reh():
        newreh_ref[0:8, :] = reh_refs[0][...]

    @pl.when(t == STEPS - 1)
    def _write_bagfeats():
        newreh_ref[0:8, :] = bf_ref[...]

    for j in range(1, NV):
        newreh_ref[8 * j:8 * (j + 1), :] = reh_refs[j][...]


def kernel(x, W1, b1, Wa1, ba1, Wa2, ba2, Wc, bc, rehearsal):
    reh_flat = rehearsal.reshape(BUFFER, L)

    grid = (B, NT)
    in_specs = [
        pl.BlockSpec((1, T, F), lambda b, n: (b, n, 0)),
        pl.BlockSpec((F, L), lambda b, n: (0, 0)),
        pl.BlockSpec((1, L), lambda b, n: (0, 0)),
        pl.BlockSpec((L, D), lambda b, n: (0, 0)),
        pl.BlockSpec((1, D), lambda b, n: (0, 0)),
        pl.BlockSpec((1, D), lambda b, n: (0, 0)),
        pl.BlockSpec((L, NUM_CLASSES), lambda b, n: (0, 0)),
        pl.BlockSpec((1, NUM_CLASSES), lambda b, n: (0, 0)),
    ] + [
        pl.BlockSpec(
            (8, L),
            (lambda j: (lambda b, n: (jnp.maximum(
                NV * ((b * NT + n + 1) % STEPS) - 1 + j, 0), 0)))(j))
        for j in range(NV)
    ]
    out_specs = [
        pl.BlockSpec((1, 1, NUM_CLASSES), lambda b, n: (b, 0, 0)),
        pl.BlockSpec((RB, L), lambda b, n: ((b * NT + n + 1) % STEPS, 0)),
    ]
    out_shapes = [
        jax.ShapeDtypeStruct((B, 1, NUM_CLASSES), jnp.float32),
        jax.ShapeDtypeStruct((BUFFER, L), jnp.float32),
    ]
    scratch_shapes = [
        pltpu.VMEM((1, L), jnp.float32),
        pltpu.SMEM((1, 1), jnp.float32),
        pltpu.SMEM((1, 1), jnp.float32),
        pltpu.VMEM((B, L), jnp.float32),
    ]

    logits, newreh = pl.pallas_call(
        _fused_kernel,
        grid=grid,
        in_specs=in_specs,
        out_specs=out_specs,
        out_shape=out_shapes,
        scratch_shapes=scratch_shapes,
        compiler_params=pltpu.CompilerParams(
            dimension_semantics=("arbitrary", "arbitrary"),
        ),
    )(x, W1.astype(jnp.bfloat16), b1.reshape(1, L),
      Wa1.astype(jnp.bfloat16), ba1.reshape(1, D), Wa2.reshape(1, D),
      Wc, bc.reshape(1, NUM_CLASSES), *([reh_flat] * NV))

    return (logits.reshape(B, NUM_CLASSES),
            newreh.reshape(NUM_CLASSES, BUFFER // NUM_CLASSES, L))


# speedup vs baseline: 1.5011x; 1.0594x over previous
import jax
import jax.numpy as jnp
from jax.experimental import pallas as pl
from jax.experimental.pallas import tpu as pltpu

B = 8
N = 8192
F = 512
L = 500
D = 128
NUM_CLASSES = 2
BUFFER = 1024

T = 4096
NT = N // T
STEPS = B * NT
RB = BUFFER // STEPS
NV = RB // 8


def _fused_kernel(x_ref, W1_ref, b1_ref, Wa1_ref, ba1_ref, Wa2t_ref,
                  Wc_ref, bc_ref, *refs):
    reh_refs = refs[:NV]
    logits_ref, newreh_ref, acc_ref, m_ref, s_ref, bf_ref = refs[NV:]
    b = pl.program_id(0)
    n = pl.program_id(1)
    t = b * NT + n

    @pl.when(n == 0)
    def _init():
        m_ref[0, 0] = -1e30
        s_ref[0, 0] = 0.0
        acc_ref[...] = jnp.zeros_like(acc_ref)

    x_t = x_ref[0]
    H = jnp.maximum(jnp.dot(x_t.astype(jnp.bfloat16), W1_ref[...],
                            preferred_element_type=jnp.float32)
                    + b1_ref[0], 0.0)
    Hb = H.astype(jnp.bfloat16)
    a = jnp.maximum(jnp.dot(Hb, Wa1_ref[...],
                            preferred_element_type=jnp.float32)
                    + ba1_ref[0], 0.0)
    sc = jax.lax.dot_general(Wa2t_ref[...], a, (((1,), (1,)), ((), ())),
                             preferred_element_type=jnp.float32)

    m_old = m_ref[0, 0]
    m_new = jnp.maximum(m_old, jnp.max(sc))
    corr = jnp.exp(m_old - m_new)
    p = jnp.exp(sc - m_new)
    m_ref[0, 0] = m_new
    s_ref[0, 0] = s_ref[0, 0] * corr + jnp.sum(p)
    acc_ref[...] = acc_ref[...] * corr + jnp.dot(
        p, H, preferred_element_type=jnp.float32)

    @pl.when(n == NT - 1)
    def _finish_bag():
        bf = acc_ref[...] / s_ref[0, 0]
        bf_ref[pl.ds(b, 1), :] = bf
        logits_ref[0] = jnp.dot(bf, Wc_ref[...],
                                preferred_element_type=jnp.float32) + bc_ref[...]

    @pl.when(t < STEPS - 1)
    def _copy_reh():
        newreh_ref[0:8, :] = reh_refs[0][...]

    @pl.when(t == STEPS - 1)
    def _write_bagfeats():
        newreh_ref[0:8, :] = bf_ref[...]

    for j in range(1, NV):
        newreh_ref[8 * j:8 * (j + 1), :] = reh_refs[j][...]


def kernel(x, W1, b1, Wa1, ba1, Wa2, ba2, Wc, bc, rehearsal):
    reh_flat = rehearsal.reshape(BUFFER, L)

    grid = (B, NT)
    in_specs = [
        pl.BlockSpec((1, T, F), lambda b, n: (b, n, 0)),
        pl.BlockSpec((F, L), lambda b, n: (0, 0)),
        pl.BlockSpec((1, L), lambda b, n: (0, 0)),
        pl.BlockSpec((L, D), lambda b, n: (0, 0)),
        pl.BlockSpec((1, D), lambda b, n: (0, 0)),
        pl.BlockSpec((1, D), lambda b, n: (0, 0)),
        pl.BlockSpec((L, NUM_CLASSES), lambda b, n: (0, 0)),
        pl.BlockSpec((1, NUM_CLASSES), lambda b, n: (0, 0)),
    ] + [
        pl.BlockSpec(
            (8, L),
            (lambda j: (lambda b, n: (jnp.maximum(
                NV * ((b * NT + n + 1) % STEPS) - 1 + j, 0), 0)))(j))
        for j in range(NV)
    ]
    out_specs = [
        pl.BlockSpec((1, 1, NUM_CLASSES), lambda b, n: (b, 0, 0)),
        pl.BlockSpec((RB, L), lambda b, n: ((b * NT + n + 1) % STEPS, 0)),
    ]
    out_shapes = [
        jax.ShapeDtypeStruct((B, 1, NUM_CLASSES), jnp.float32),
        jax.ShapeDtypeStruct((BUFFER, L), jnp.float32),
    ]
    scratch_shapes = [
        pltpu.VMEM((1, L), jnp.float32),
        pltpu.SMEM((1, 1), jnp.float32),
        pltpu.SMEM((1, 1), jnp.float32),
        pltpu.VMEM((B, L), jnp.float32),
    ]

    logits, newreh = pl.pallas_call(
        _fused_kernel,
        grid=grid,
        in_specs=in_specs,
        out_specs=out_specs,
        out_shape=out_shapes,
        scratch_shapes=scratch_shapes,
        compiler_params=pltpu.CompilerParams(
            dimension_semantics=("arbitrary", "arbitrary"),
        ),
    )(x, W1.astype(jnp.bfloat16), b1.reshape(1, L),
      Wa1.astype(jnp.bfloat16), ba1.reshape(1, D), Wa2.reshape(1, D),
      Wc, bc.reshape(1, NUM_CLASSES), *([reh_flat] * NV))

    return (logits.reshape(B, NUM_CLASSES),
            newreh.reshape(NUM_CLASSES, BUFFER // NUM_CLASSES, L))


# speedup vs baseline: 1.5573x; 1.0375x over previous
import jax
import jax.numpy as jnp
from jax.experimental import pallas as pl
from jax.experimental.pallas import tpu as pltpu

B = 8
N = 8192
F = 512
L = 500
D = 128
NUM_CLASSES = 2
BUFFER = 1024

T = 8192
NT = N // T
STEPS = B * NT
RB = BUFFER // STEPS
NV = RB // 8


def _fused_kernel(x_ref, W1_ref, b1_ref, Wa1_ref, ba1_ref, Wa2t_ref,
                  Wc_ref, bc_ref, *refs):
    reh_refs = refs[:NV]
    logits_ref, newreh_ref, acc_ref, m_ref, s_ref, bf_ref = refs[NV:]
    b = pl.program_id(0)
    n = pl.program_id(1)
    t = b * NT + n

    @pl.when(n == 0)
    def _init():
        m_ref[0, 0] = -1e30
        s_ref[0, 0] = 0.0
        acc_ref[...] = jnp.zeros_like(acc_ref)

    x_t = x_ref[0]
    H = jnp.maximum(jnp.dot(x_t.astype(jnp.bfloat16), W1_ref[...],
                            preferred_element_type=jnp.float32)
                    + b1_ref[0], 0.0)
    Hb = H.astype(jnp.bfloat16)
    a = jnp.maximum(jnp.dot(Hb, Wa1_ref[...],
                            preferred_element_type=jnp.float32)
                    + ba1_ref[0], 0.0)
    sc = jax.lax.dot_general(Wa2t_ref[...], a, (((1,), (1,)), ((), ())),
                             preferred_element_type=jnp.float32)

    m_old = m_ref[0, 0]
    m_new = jnp.maximum(m_old, jnp.max(sc))
    corr = jnp.exp(m_old - m_new)
    p = jnp.exp(sc - m_new)
    m_ref[0, 0] = m_new
    s_ref[0, 0] = s_ref[0, 0] * corr + jnp.sum(p)
    acc_ref[...] = acc_ref[...] * corr + jnp.dot(
        p, H, preferred_element_type=jnp.float32)

    @pl.when(n == NT - 1)
    def _finish_bag():
        bf = acc_ref[...] / s_ref[0, 0]
        bf_ref[pl.ds(b, 1), :] = bf
        logits_ref[0] = jnp.dot(bf, Wc_ref[...],
                                preferred_element_type=jnp.float32) + bc_ref[...]

    @pl.when(t < STEPS - 1)
    def _copy_reh():
        newreh_ref[0:8, :] = reh_refs[0][...]

    @pl.when(t == STEPS - 1)
    def _write_bagfeats():
        newreh_ref[0:8, :] = bf_ref[...]

    for j in range(1, NV):
        newreh_ref[8 * j:8 * (j + 1), :] = reh_refs[j][...]


def kernel(x, W1, b1, Wa1, ba1, Wa2, ba2, Wc, bc, rehearsal):
    reh_flat = rehearsal.reshape(BUFFER, L)

    grid = (B, NT)
    in_specs = [
        pl.BlockSpec((1, T, F), lambda b, n: (b, n, 0)),
        pl.BlockSpec((F, L), lambda b, n: (0, 0)),
        pl.BlockSpec((1, L), lambda b, n: (0, 0)),
        pl.BlockSpec((L, D), lambda b, n: (0, 0)),
        pl.BlockSpec((1, D), lambda b, n: (0, 0)),
        pl.BlockSpec((1, D), lambda b, n: (0, 0)),
        pl.BlockSpec((L, NUM_CLASSES), lambda b, n: (0, 0)),
        pl.BlockSpec((1, NUM_CLASSES), lambda b, n: (0, 0)),
    ] + [
        pl.BlockSpec(
            (8, L),
            (lambda j: (lambda b, n: (jnp.maximum(
                NV * ((b * NT + n + 1) % STEPS) - 1 + j, 0), 0)))(j))
        for j in range(NV)
    ]
    out_specs = [
        pl.BlockSpec((1, 1, NUM_CLASSES), lambda b, n: (b, 0, 0)),
        pl.BlockSpec((RB, L), lambda b, n: ((b * NT + n + 1) % STEPS, 0)),
    ]
    out_shapes = [
        jax.ShapeDtypeStruct((B, 1, NUM_CLASSES), jnp.float32),
        jax.ShapeDtypeStruct((BUFFER, L), jnp.float32),
    ]
    scratch_shapes = [
        pltpu.VMEM((1, L), jnp.float32),
        pltpu.SMEM((1, 1), jnp.float32),
        pltpu.SMEM((1, 1), jnp.float32),
        pltpu.VMEM((B, L), jnp.float32),
    ]

    logits, newreh = pl.pallas_call(
        _fused_kernel,
        grid=grid,
        in_specs=in_specs,
        out_specs=out_specs,
        out_shape=out_shapes,
        scratch_shapes=scratch_shapes,
        compiler_params=pltpu.CompilerParams(
            dimension_semantics=("arbitrary", "arbitrary"),
        ),
    )(x, W1.astype(jnp.bfloat16), b1.reshape(1, L),
      Wa1.astype(jnp.bfloat16), ba1.reshape(1, D), Wa2.reshape(1, D),
      Wc, bc.reshape(1, NUM_CLASSES), *([reh_flat] * NV))

    return (logits.reshape(B, NUM_CLASSES),
            newreh.reshape(NUM_CLASSES, BUFFER // NUM_CLASSES, L))


# speedup vs baseline: 1.5576x; 1.0002x over previous
import jax
import jax.numpy as jnp
from jax.experimental import pallas as pl
from jax.experimental.pallas import tpu as pltpu

B = 8
N = 8192
F = 512
L = 500
D = 128
NUM_CLASSES = 2
BUFFER = 1024

T = 8192
NT = N // T
STEPS = B * NT
RB = BUFFER // STEPS
NV = RB // 8


def _fused_kernel(x_ref, W1_ref, b1_ref, Wa1_ref, ba1_ref, Wa2t_ref,
                  Wc_ref, bc_ref, *refs):
    reh_refs = refs[:NV]
    logits_ref, newreh_ref, acc_ref, m_ref, s_ref, bf_ref = refs[NV:]
    b = pl.program_id(0)
    n = pl.program_id(1)
    t = b * NT + n

    @pl.when(n == 0)
    def _init():
        m_ref[0, 0] = -1e30
        s_ref[0, 0] = 0.0
        acc_ref[...] = jnp.zeros_like(acc_ref)

    x_t = x_ref[0]
    H = jnp.maximum(jnp.dot(x_t.astype(jnp.bfloat16), W1_ref[...],
                            preferred_element_type=jnp.float32)
                    + b1_ref[0], 0.0)
    Hb = H.astype(jnp.bfloat16)
    a = jnp.maximum(jnp.dot(Hb, Wa1_ref[...],
                            preferred_element_type=jnp.float32)
                    + ba1_ref[0], 0.0)
    sc = jax.lax.dot_general(Wa2t_ref[...], a, (((1,), (1,)), ((), ())),
                             preferred_element_type=jnp.float32)

    m_old = m_ref[0, 0]
    m_new = jnp.maximum(m_old, jnp.max(sc))
    corr = jnp.exp(m_old - m_new)
    p = jnp.exp(sc - m_new)
    m_ref[0, 0] = m_new
    s_ref[0, 0] = s_ref[0, 0] * corr + jnp.sum(p)
    acc_ref[...] = acc_ref[...] * corr + jnp.dot(
        p.astype(jnp.bfloat16), Hb,
        preferred_element_type=jnp.float32)

    @pl.when(n == NT - 1)
    def _finish_bag():
        bf = acc_ref[...] / s_ref[0, 0]
        bf_ref[pl.ds(b, 1), :] = bf
        logits_ref[0] = jnp.dot(bf, Wc_ref[...],
                                preferred_element_type=jnp.float32) + bc_ref[...]

    @pl.when(t < STEPS - 1)
    def _copy_reh():
        newreh_ref[0:8, :] = reh_refs[0][...]

    @pl.when(t == STEPS - 1)
    def _write_bagfeats():
        newreh_ref[0:8, :] = bf_ref[...]

    for j in range(1, NV):
        newreh_ref[8 * j:8 * (j + 1), :] = reh_refs[j][...]


def kernel(x, W1, b1, Wa1, ba1, Wa2, ba2, Wc, bc, rehearsal):
    reh_flat = rehearsal.reshape(BUFFER, L)

    grid = (B, NT)
    in_specs = [
        pl.BlockSpec((1, T, F), lambda b, n: (b, n, 0)),
        pl.BlockSpec((F, L), lambda b, n: (0, 0)),
        pl.BlockSpec((1, L), lambda b, n: (0, 0)),
        pl.BlockSpec((L, D), lambda b, n: (0, 0)),
        pl.BlockSpec((1, D), lambda b, n: (0, 0)),
        pl.BlockSpec((1, D), lambda b, n: (0, 0)),
        pl.BlockSpec((L, NUM_CLASSES), lambda b, n: (0, 0)),
        pl.BlockSpec((1, NUM_CLASSES), lambda b, n: (0, 0)),
    ] + [
        pl.BlockSpec(
            (8, L),
            (lambda j: (lambda b, n: (jnp.maximum(
                NV * ((b * NT + n + 1) % STEPS) - 1 + j, 0), 0)))(j))
        for j in range(NV)
    ]
    out_specs = [
        pl.BlockSpec((1, 1, NUM_CLASSES), lambda b, n: (b, 0, 0)),
        pl.BlockSpec((RB, L), lambda b, n: ((b * NT + n + 1) % STEPS, 0)),
    ]
    out_shapes = [
        jax.ShapeDtypeStruct((B, 1, NUM_CLASSES), jnp.float32),
        jax.ShapeDtypeStruct((BUFFER, L), jnp.float32),
    ]
    scratch_shapes = [
        pltpu.VMEM((1, L), jnp.float32),
        pltpu.SMEM((1, 1), jnp.float32),
        pltpu.SMEM((1, 1), jnp.float32),
        pltpu.VMEM((B, L), jnp.float32),
    ]

    logits, newreh = pl.pallas_call(
        _fused_kernel,
        grid=grid,
        in_specs=in_specs,
        out_specs=out_specs,
        out_shape=out_shapes,
        scratch_shapes=scratch_shapes,
        compiler_params=pltpu.CompilerParams(
            dimension_semantics=("arbitrary", "arbitrary"),
        ),
    )(x, W1.astype(jnp.bfloat16), b1.reshape(1, L),
      Wa1.astype(jnp.bfloat16), ba1.reshape(1, D), Wa2.reshape(1, D),
      Wc, bc.reshape(1, NUM_CLASSES), *([reh_flat] * NV))

    return (logits.reshape(B, NUM_CLASSES),
            newreh.reshape(NUM_CLASSES, BUFFER // NUM_CLASSES, L))


# speedup vs baseline: 1.5581x; 1.0003x over previous
import jax
import jax.numpy as jnp
from jax.experimental import pallas as pl
from jax.experimental.pallas import tpu as pltpu

B = 8
N = 8192
F = 512
L = 500
D = 128
NUM_CLASSES = 2
BUFFER = 1024

T = 8192
NT = N // T
STEPS = B * NT
RB = BUFFER // STEPS
NV = RB // 8


def _fused_kernel(x_ref, W1_ref, b1_ref, Wa1_ref, ba1_ref, Wa2t_ref,
                  Wc_ref, bc_ref, *refs):
    reh_refs = refs[:NV]
    logits_ref, newreh_ref, acc_ref, m_ref, s_ref, bf_ref = refs[NV:]
    b = pl.program_id(0)
    n = pl.program_id(1)
    t = b * NT + n

    @pl.when(n == 0)
    def _init():
        m_ref[0, 0] = -1e30
        s_ref[0, 0] = 0.0
        acc_ref[...] = jnp.zeros_like(acc_ref)

    x_t = x_ref[0]
    H = jnp.maximum(jnp.dot(x_t.astype(jnp.bfloat16), W1_ref[...],
                            preferred_element_type=jnp.float32)
                    + b1_ref[0], 0.0)
    Hb = H.astype(jnp.bfloat16)
    a = jnp.maximum(jnp.dot(Hb, Wa1_ref[...],
                            preferred_element_type=jnp.float32)
                    + ba1_ref[0], 0.0)
    sc = jax.lax.dot_general(Wa2t_ref[...], a, (((1,), (1,)), ((), ())),
                             preferred_element_type=jnp.float32)

    m_old = m_ref[0, 0]
    m_new = jnp.maximum(m_old, jnp.max(sc))
    corr = jnp.exp(m_old - m_new)
    p = jnp.exp(sc - m_new)
    m_ref[0, 0] = m_new
    s_ref[0, 0] = s_ref[0, 0] * corr + jnp.sum(p)
    acc_ref[...] = acc_ref[...] * corr + jnp.dot(
        p, H, preferred_element_type=jnp.float32)

    @pl.when(n == NT - 1)
    def _finish_bag():
        bf = acc_ref[...] / s_ref[0, 0]
        bf_ref[pl.ds(b, 1), :] = bf
        logits_ref[0] = jnp.dot(bf, Wc_ref[...],
                                preferred_element_type=jnp.float32) + bc_ref[...]

    @pl.when(t < STEPS - 1)
    def _copy_reh():
        newreh_ref[0:8, :] = reh_refs[0][...]

    @pl.when(t == STEPS - 1)
    def _write_bagfeats():
        newreh_ref[0:8, :] = bf_ref[...]

    for j in range(1, NV):
        newreh_ref[8 * j:8 * (j + 1), :] = reh_refs[j][...]


def kernel(x, W1, b1, Wa1, ba1, Wa2, ba2, Wc, bc, rehearsal):
    reh_flat = rehearsal.reshape(BUFFER, L)

    grid = (B, NT)
    in_specs = [
        pl.BlockSpec((1, T, F), lambda b, n: (b, n, 0)),
        pl.BlockSpec((F, L), lambda b, n: (0, 0)),
        pl.BlockSpec((1, L), lambda b, n: (0, 0)),
        pl.BlockSpec((L, D), lambda b, n: (0, 0)),
        pl.BlockSpec((1, D), lambda b, n: (0, 0)),
        pl.BlockSpec((1, D), lambda b, n: (0, 0)),
        pl.BlockSpec((L, NUM_CLASSES), lambda b, n: (0, 0)),
        pl.BlockSpec((1, NUM_CLASSES), lambda b, n: (0, 0)),
    ] + [
        pl.BlockSpec(
            (8, L),
            (lambda j: (lambda b, n: (jnp.maximum(
                NV * ((b * NT + n + 1) % STEPS) - 1 + j, 0), 0)))(j))
        for j in range(NV)
    ]
    out_specs = [
        pl.BlockSpec((1, 1, NUM_CLASSES), lambda b, n: (b, 0, 0)),
        pl.BlockSpec((RB, L), lambda b, n: ((b * NT + n + 1) % STEPS, 0)),
    ]
    out_shapes = [
        jax.ShapeDtypeStruct((B, 1, NUM_CLASSES), jnp.float32),
        jax.ShapeDtypeStruct((BUFFER, L), jnp.float32),
    ]
    scratch_shapes = [
        pltpu.VMEM((1, L), jnp.float32),
        pltpu.SMEM((1, 1), jnp.float32),
        pltpu.SMEM((1, 1), jnp.float32),
        pltpu.VMEM((B, L), jnp.float32),
    ]

    logits, newreh = pl.pallas_call(
        _fused_kernel,
        grid=grid,
        in_specs=in_specs,
        out_specs=out_specs,
        out_shape=out_shapes,
        scratch_shapes=scratch_shapes,
        compiler_params=pltpu.CompilerParams(
            dimension_semantics=("arbitrary", "arbitrary"),
        ),
    )(x, W1.astype(jnp.bfloat16), b1.reshape(1, L),
      Wa1.astype(jnp.bfloat16), ba1.reshape(1, D), Wa2.reshape(1, D),
      Wc, bc.reshape(1, NUM_CLASSES), *([reh_flat] * NV))

    return (logits.reshape(B, NUM_CLASSES),
            newreh.reshape(NUM_CLASSES, BUFFER // NUM_CLASSES, L))
